# baseline (speedup 1.0000x reference)
import functools

import jax
import jax.numpy as jnp
from jax.experimental import pallas as pl
from jax.experimental.pallas import tpu as pltpu

D_ROWS = 64
SIZE = 1_000_000
BC = 8192
NBLK = 123
PADT = NBLK * BC


def _body(ptr_ref, delays_ref, spikes_ref, buf_ref, out_ref):
    p = ptr_ref[0]
    d = delays_ref[0, 0, :]
    r = (p + 1 - d) & (D_ROWS - 1)
    rows = jax.lax.broadcasted_iota(jnp.int32, (D_ROWS, BC), 0)
    hit = rows == r[None, :]
    val = jnp.sum(jnp.where(hit, buf_ref[...], 0.0), axis=0)
    out_ref[0, 0, :] = jnp.where(r == (p & (D_ROWS - 1)),
                                 spikes_ref[0, 0, :], val)


@jax.jit
def _tc_delay_gather(buffer, spikes_p, delays_p, ptr_arr):
    grid = (NBLK,)
    return pl.pallas_call(
        _body,
        grid=grid,
        in_specs=[
            pl.BlockSpec(memory_space=pltpu.SMEM),
            pl.BlockSpec((1, 1, BC), lambda i: (i, 0, 0)),
            pl.BlockSpec((1, 1, BC), lambda i: (i, 0, 0)),
            pl.BlockSpec((D_ROWS, BC), lambda i: (0, i)),
        ],
        out_specs=pl.BlockSpec((1, 1, BC), lambda i: (i, 0, 0)),
        out_shape=jax.ShapeDtypeStruct((NBLK, 1, BC), jnp.float32),
    )(ptr_arr, delays_p, spikes_p, buffer)


def kernel(buffer, spikes, delays, ptr):
    pad = PADT - SIZE
    delays_p = jnp.concatenate(
        [delays.astype(jnp.int32), jnp.zeros((pad,), jnp.int32)]
    ).reshape(NBLK, 1, BC)
    spikes_p = jnp.concatenate(
        [spikes.astype(jnp.float32), jnp.zeros((pad,), jnp.float32)]
    ).reshape(NBLK, 1, BC)
    ptr_arr = jnp.full((1,), ptr, dtype=jnp.int32)
    out = _tc_delay_gather(buffer, spikes_p, delays_p, ptr_arr)
    return out.reshape(-1)[:SIZE]


# speedup vs baseline: 1.1715x; 1.1715x over previous
import functools

import jax
import jax.numpy as jnp
from jax import lax
from jax.experimental import pallas as pl
from jax.experimental.pallas import tpu as pltpu
from jax.experimental.pallas import tpu_sc as plsc

D_ROWS = 64
SIZE = 1_000_000
NC, NS, L = 2, 16, 16
NW = NC * NS

W = 512
NWIN = 30
TSPAN = W * NWIN
SC_COLS = NW * TSPAN

BC = 8192
BOFF = SC_COLS // BC
NBLK = 63
TC_PAD = BOFF * BC + NBLK * BC - SIZE

_mesh = plsc.VectorSubcoreMesh(core_axis_name="c", subcore_axis_name="s")


@functools.partial(
    pl.kernel,
    out_type=jax.ShapeDtypeStruct((SC_COLS,), jnp.float32),
    mesh=_mesh,
    scratch_types=[
        pltpu.VMEM((D_ROWS, W), jnp.float32),
        pltpu.VMEM((D_ROWS, W), jnp.float32),
        pltpu.VMEM((W,), jnp.int32),
        pltpu.VMEM((W,), jnp.int32),
        pltpu.VMEM((W,), jnp.float32),
        pltpu.VMEM((W,), jnp.float32),
        pltpu.VMEM((W,), jnp.float32),
        pltpu.VMEM((W,), jnp.float32),
        pltpu.VMEM((L,), jnp.int32),
        pltpu.SemaphoreType.DMA,
        pltpu.SemaphoreType.DMA,
    ],
)
def _sc_delay_gather(buf_hbm, delays_hbm, spikes_hbm, ptr_hbm, out_hbm,
                     blkA, blkB, dvA, dvB, svA, svB, ovA, ovB,
                     pv, sem_in, sem_out):
    blk = (blkA, blkB)
    dv = (dvA, dvB)
    sv = (svA, svB)
    ov = (ovA, ovB)
    wid = lax.axis_index("s") * NC + lax.axis_index("c")
    tbase = wid * TSPAN
    pltpu.sync_copy(ptr_hbm, pv)
    ptr_v = pv[...]
    p1 = ptr_v + 1
    ptr_mod = ptr_v & (D_ROWS - 1)

    def cbase_of(k):
        return pl.multiple_of(tbase + k * W, W)

    def fire_in(k, b):
        cbase = cbase_of(k)
        pltpu.async_copy(buf_hbm.at[:, pl.ds(cbase, W)], blk[b], sem_in)
        pltpu.async_copy(delays_hbm.at[pl.ds(cbase, W)], dv[b], sem_in)
        pltpu.async_copy(spikes_hbm.at[pl.ds(cbase, W)], sv[b], sem_in)

    def wait_in(k, b):
        cbase = cbase_of(k)
        pltpu.make_async_copy(buf_hbm.at[:, pl.ds(cbase, W)], blk[b],
                              sem_in).wait()
        pltpu.make_async_copy(delays_hbm.at[pl.ds(cbase, W)], dv[b],
                              sem_in).wait()
        pltpu.make_async_copy(spikes_hbm.at[pl.ds(cbase, W)], sv[b],
                              sem_in).wait()

    def select_tree(blk_ref, d, off):
        r = (p1 - d) & (D_ROWS - 1)
        vals = [blk_ref[row, pl.ds(off, L)] for row in range(D_ROWS)]
        for bit in range(6):
            take_hi = ((r >> bit) & 1) == 1
            vals = [jnp.where(take_hi, vals[2 * i + 1], vals[2 * i])
                    for i in range(len(vals) // 2)]
        return r, vals[0]

    def compute(k, b):
        def group(u, carry):
            off = u * L
            d = dv[b][pl.ds(off, L)]
            r, val = select_tree(blk[b], d, off)
            ov[b][pl.ds(off, L)] = jnp.where(
                r == ptr_mod, sv[b][pl.ds(off, L)], val)
            return carry
        lax.fori_loop(0, W // L, group, 0)

    def fire_out(k, b):
        pltpu.async_copy(ov[b], out_hbm.at[pl.ds(cbase_of(k), W)], sem_out)

    def wait_out():
        pltpu.make_async_copy(ov[0], out_hbm.at[pl.ds(tbase, W)],
                              sem_out).wait()

    fire_in(0, 0)

    def body(j, carry):
        k0 = 2 * j
        k1 = k0 + 1
        fire_in(k1, 1)
        wait_in(k0, 0)

        @pl.when(j >= 1)
        def _():
            wait_out()

        compute(k0, 0)
        fire_out(k0, 0)

        @pl.when(k1 + 1 < NWIN)
        def _():
            fire_in(k0 + 2, 0)

        wait_in(k1, 1)

        @pl.when(j >= 1)
        def _():
            wait_out()

        compute(k1, 1)
        fire_out(k1, 1)
        return carry

    lax.fori_loop(0, NWIN // 2, body, 0)
    wait_out()
    wait_out()


def _tc_body(ptr_ref, delays_ref, spikes_ref, buf_ref, out_ref):
    p = ptr_ref[0]
    d = delays_ref[0, 0, :]
    r = (p + 1 - d) & (D_ROWS - 1)
    rows = jax.lax.broadcasted_iota(jnp.int32, (D_ROWS, BC), 0)
    hit = rows == r[None, :]
    val = jnp.sum(jnp.where(hit, buf_ref[...], 0.0), axis=0)
    out_ref[0, 0, :] = jnp.where(r == (p & (D_ROWS - 1)),
                                 spikes_ref[0, 0, :], val)


def _tc_delay_gather(buffer, delays_tc, spikes_tc, ptr_arr):
    return pl.pallas_call(
        _tc_body,
        grid=(NBLK,),
        in_specs=[
            pl.BlockSpec(memory_space=pltpu.SMEM),
            pl.BlockSpec((1, 1, BC), lambda i: (i, 0, 0)),
            pl.BlockSpec((1, 1, BC), lambda i: (i, 0, 0)),
            pl.BlockSpec((D_ROWS, BC), lambda i: (0, i + BOFF)),
        ],
        out_specs=pl.BlockSpec((1, 1, BC), lambda i: (i, 0, 0)),
        out_shape=jax.ShapeDtypeStruct((NBLK, 1, BC), jnp.float32),
    )(ptr_arr, delays_tc, spikes_tc, buffer)


def kernel(buffer, spikes, delays, ptr):
    delays_i = delays.astype(jnp.int32)
    spikes_f = spikes.astype(jnp.float32)
    ptr_sc = jnp.full((L,), ptr, dtype=jnp.int32)
    ptr_tc = jnp.full((1,), ptr, dtype=jnp.int32)
    delays_tc = jnp.concatenate(
        [delays_i[SC_COLS:], jnp.zeros((TC_PAD,), jnp.int32)]
    ).reshape(NBLK, 1, BC)
    spikes_tc = jnp.concatenate(
        [spikes_f[SC_COLS:], jnp.zeros((TC_PAD,), jnp.float32)]
    ).reshape(NBLK, 1, BC)
    out_sc = _sc_delay_gather(buffer, delays_i, spikes_f, ptr_sc)
    out_tc = _tc_delay_gather(buffer, delays_tc, spikes_tc, ptr_tc)
    return jnp.concatenate([out_sc, out_tc.reshape(-1)[:SIZE - SC_COLS]])


# speedup vs baseline: 1.1774x; 1.0050x over previous
import functools

import jax
import jax.numpy as jnp
from jax import lax
from jax.experimental import pallas as pl
from jax.experimental.pallas import tpu as pltpu
from jax.experimental.pallas import tpu_sc as plsc

D_ROWS = 64
SIZE = 1_000_000
NC, NS, L = 2, 16, 16
NW = NC * NS

W = 512
NWIN = 30
TSPAN = W * NWIN
SC_COLS = NW * TSPAN

BC = 8192
BOFF = SC_COLS // BC
NBLK = 63
TC_PAD = BOFF * BC + NBLK * BC - SIZE

_mesh = plsc.VectorSubcoreMesh(core_axis_name="c", subcore_axis_name="s")


@functools.partial(
    pl.kernel,
    out_type=jax.ShapeDtypeStruct((SC_COLS,), jnp.float32),
    mesh=_mesh,
    scratch_types=[
        pltpu.VMEM((D_ROWS, W), jnp.float32),
        pltpu.VMEM((D_ROWS, W), jnp.float32),
        pltpu.VMEM((W,), jnp.int32),
        pltpu.VMEM((W,), jnp.int32),
        pltpu.VMEM((W,), jnp.float32),
        pltpu.VMEM((W,), jnp.float32),
        pltpu.VMEM((W,), jnp.float32),
        pltpu.VMEM((W,), jnp.float32),
        pltpu.VMEM((L,), jnp.int32),
        pltpu.SemaphoreType.DMA,
        pltpu.SemaphoreType.DMA,
    ],
)
def _sc_delay_gather(buf_hbm, delays_hbm, spikes_hbm, ptr_hbm, out_hbm,
                     blkA, blkB, dvA, dvB, svA, svB, ovA, ovB,
                     pv, sem_in, sem_out):
    blk = (blkA, blkB)
    dv = (dvA, dvB)
    sv = (svA, svB)
    ov = (ovA, ovB)
    wid = lax.axis_index("s") * NC + lax.axis_index("c")
    tbase = wid * TSPAN
    pltpu.sync_copy(ptr_hbm, pv)
    ptr_v = pv[...]
    p1 = ptr_v + 1
    ptr_mod = ptr_v & (D_ROWS - 1)

    def cbase_of(k):
        return pl.multiple_of(tbase + k * W, W)

    def fire_in(k, b):
        cbase = cbase_of(k)
        pltpu.async_copy(buf_hbm.at[:, pl.ds(cbase, W)], blk[b], sem_in)
        pltpu.async_copy(delays_hbm.at[pl.ds(cbase, W)], dv[b], sem_in)
        pltpu.async_copy(spikes_hbm.at[pl.ds(cbase, W)], sv[b], sem_in)

    def wait_in(k, b):
        cbase = cbase_of(k)
        pltpu.make_async_copy(buf_hbm.at[:, pl.ds(cbase, W)], blk[b],
                              sem_in).wait()
        pltpu.make_async_copy(delays_hbm.at[pl.ds(cbase, W)], dv[b],
                              sem_in).wait()
        pltpu.make_async_copy(spikes_hbm.at[pl.ds(cbase, W)], sv[b],
                              sem_in).wait()

    def select_tree(blk_ref, d, off):
        r = (p1 - d) & (D_ROWS - 1)
        vals = [blk_ref[row, pl.ds(off, L)] for row in range(D_ROWS)]
        for bit in range(6):
            take_hi = ((r >> bit) & 1) == 1
            vals = [jnp.where(take_hi, vals[2 * i + 1], vals[2 * i])
                    for i in range(len(vals) // 2)]
        return r, vals[0]

    def compute(k, b):
        def group(u, carry):
            off = u * L
            d = dv[b][pl.ds(off, L)]
            r, val = select_tree(blk[b], d, off)
            ov[b][pl.ds(off, L)] = jnp.where(
                r == ptr_mod, sv[b][pl.ds(off, L)], val)
            return carry
        lax.fori_loop(0, W // L, group, 0)

    def fire_out(k, b):
        pltpu.async_copy(ov[b], out_hbm.at[pl.ds(cbase_of(k), W)], sem_out)

    def wait_out():
        pltpu.make_async_copy(ov[0], out_hbm.at[pl.ds(tbase, W)],
                              sem_out).wait()

    fire_in(0, 0)

    def body(j, carry):
        k0 = 2 * j
        k1 = k0 + 1
        fire_in(k1, 1)
        wait_in(k0, 0)

        @pl.when(j >= 1)
        def _():
            wait_out()

        compute(k0, 0)
        fire_out(k0, 0)

        @pl.when(k1 + 1 < NWIN)
        def _():
            fire_in(k0 + 2, 0)

        wait_in(k1, 1)

        @pl.when(j >= 1)
        def _():
            wait_out()

        compute(k1, 1)
        fire_out(k1, 1)
        return carry

    lax.fori_loop(0, NWIN // 2, body, 0)
    wait_out()
    wait_out()


def _tc_body(ptr_ref, delays_ref, spikes_ref, buf_ref, out_ref):
    p = ptr_ref[0]
    d = delays_ref[0, 0, :]
    r = (p + 1 - d) & (D_ROWS - 1)
    rows = jax.lax.broadcasted_iota(jnp.int32, (D_ROWS, BC), 0)
    hit = rows == r[None, :]
    masked = jnp.where(hit, buf_ref[...], 0.0)
    ones = jnp.ones((1, D_ROWS), jnp.float32)
    val = jax.lax.dot_general(
        ones, masked, (((1,), (0,)), ((), ())),
        preferred_element_type=jnp.float32)[0]
    out_ref[0, 0, :] = jnp.where(r == (p & (D_ROWS - 1)),
                                 spikes_ref[0, 0, :], val)


def _tc_delay_gather(buffer, delays_tc, spikes_tc, ptr_arr):
    return pl.pallas_call(
        _tc_body,
        grid=(NBLK,),
        in_specs=[
            pl.BlockSpec(memory_space=pltpu.SMEM),
            pl.BlockSpec((1, 1, BC), lambda i: (i, 0, 0)),
            pl.BlockSpec((1, 1, BC), lambda i: (i, 0, 0)),
            pl.BlockSpec((D_ROWS, BC), lambda i: (0, i + BOFF)),
        ],
        out_specs=pl.BlockSpec((1, 1, BC), lambda i: (i, 0, 0)),
        out_shape=jax.ShapeDtypeStruct((NBLK, 1, BC), jnp.float32),
    )(ptr_arr, delays_tc, spikes_tc, buffer)


def kernel(buffer, spikes, delays, ptr):
    delays_i = delays.astype(jnp.int32)
    spikes_f = spikes.astype(jnp.float32)
    ptr_sc = jnp.full((L,), ptr, dtype=jnp.int32)
    ptr_tc = jnp.full((1,), ptr, dtype=jnp.int32)
    delays_tc = jnp.concatenate(
        [delays_i[SC_COLS:], jnp.zeros((TC_PAD,), jnp.int32)]
    ).reshape(NBLK, 1, BC)
    spikes_tc = jnp.concatenate(
        [spikes_f[SC_COLS:], jnp.zeros((TC_PAD,), jnp.float32)]
    ).reshape(NBLK, 1, BC)
    out_sc = _sc_delay_gather(buffer, delays_i, spikes_f, ptr_sc)
    out_tc = _tc_delay_gather(buffer, delays_tc, spikes_tc, ptr_tc)
    return jnp.concatenate([out_sc, out_tc.reshape(-1)[:SIZE - SC_COLS]])


# speedup vs baseline: 1.2459x; 1.0581x over previous
import functools

import jax
import jax.numpy as jnp
from jax import lax
from jax.experimental import pallas as pl
from jax.experimental.pallas import tpu as pltpu
from jax.experimental.pallas import tpu_sc as plsc

D_ROWS = 64
SIZE = 1_000_000
NC, NS, L = 2, 16, 16
NW = NC * NS

W = 512
NWIN = 30
TSPAN = W * NWIN
SC_COLS = NW * TSPAN

BC = 8192
BOFF = SC_COLS // BC
NBLK = 63
TC_PAD = BOFF * BC + NBLK * BC - SIZE

_mesh = plsc.VectorSubcoreMesh(core_axis_name="c", subcore_axis_name="s")


@functools.partial(
    pl.kernel,
    out_type=jax.ShapeDtypeStruct((SC_COLS,), jnp.float32),
    mesh=_mesh,
    scratch_types=[
        pltpu.VMEM((D_ROWS, W), jnp.float32),
        pltpu.VMEM((D_ROWS, W), jnp.float32),
        pltpu.VMEM((W,), jnp.int32),
        pltpu.VMEM((W,), jnp.int32),
        pltpu.VMEM((W,), jnp.float32),
        pltpu.VMEM((W,), jnp.float32),
        pltpu.VMEM((W,), jnp.float32),
        pltpu.VMEM((W,), jnp.float32),
        pltpu.VMEM((L,), jnp.int32),
        pltpu.SemaphoreType.DMA,
        pltpu.SemaphoreType.DMA,
    ],
)
def _sc_delay_gather(buf_hbm, delays_hbm, spikes_hbm, ptr_hbm, out_hbm,
                     blkA, blkB, dvA, dvB, svA, svB, ovA, ovB,
                     pv, sem_in, sem_out):
    blk = (blkA, blkB)
    dv = (dvA, dvB)
    sv = (svA, svB)
    ov = (ovA, ovB)
    wid = lax.axis_index("s") * NC + lax.axis_index("c")
    tbase = wid * TSPAN
    pltpu.sync_copy(ptr_hbm, pv)
    ptr_v = pv[...]
    p1 = ptr_v + 1
    ptr_mod = ptr_v & (D_ROWS - 1)

    def cbase_of(k):
        return pl.multiple_of(tbase + k * W, W)

    def fire_in(k, b):
        cbase = cbase_of(k)
        pltpu.async_copy(buf_hbm.at[:, pl.ds(cbase, W)], blk[b], sem_in)
        pltpu.async_copy(delays_hbm.at[pl.ds(cbase, W)], dv[b], sem_in)
        pltpu.async_copy(spikes_hbm.at[pl.ds(cbase, W)], sv[b], sem_in)

    def wait_in(k, b):
        cbase = cbase_of(k)
        pltpu.make_async_copy(buf_hbm.at[:, pl.ds(cbase, W)], blk[b],
                              sem_in).wait()
        pltpu.make_async_copy(delays_hbm.at[pl.ds(cbase, W)], dv[b],
                              sem_in).wait()
        pltpu.make_async_copy(spikes_hbm.at[pl.ds(cbase, W)], sv[b],
                              sem_in).wait()

    def select_tree(blk_ref, d, off):
        r = (p1 - d) & (D_ROWS - 1)
        vals = [blk_ref[row, pl.ds(off, L)] for row in range(D_ROWS)]
        for bit in range(6):
            take_hi = ((r >> bit) & 1) == 1
            vals = [jnp.where(take_hi, vals[2 * i + 1], vals[2 * i])
                    for i in range(len(vals) // 2)]
        return r, vals[0]

    def compute(k, b):
        def group(u, carry):
            off = u * L
            d = dv[b][pl.ds(off, L)]
            r, val = select_tree(blk[b], d, off)
            ov[b][pl.ds(off, L)] = jnp.where(
                r == ptr_mod, sv[b][pl.ds(off, L)], val)
            return carry
        lax.fori_loop(0, W // L, group, 0)

    def fire_out(k, b):
        pltpu.async_copy(ov[b], out_hbm.at[pl.ds(cbase_of(k), W)], sem_out)

    def wait_out():
        pltpu.make_async_copy(ov[0], out_hbm.at[pl.ds(tbase, W)],
                              sem_out).wait()

    fire_in(0, 0)

    def body(j, carry):
        k0 = 2 * j
        k1 = k0 + 1
        fire_in(k1, 1)
        wait_in(k0, 0)

        @pl.when(j >= 1)
        def _():
            wait_out()

        compute(k0, 0)
        fire_out(k0, 0)

        @pl.when(k1 + 1 < NWIN)
        def _():
            fire_in(k0 + 2, 0)

        wait_in(k1, 1)

        @pl.when(j >= 1)
        def _():
            wait_out()

        compute(k1, 1)
        fire_out(k1, 1)
        return carry

    lax.fori_loop(0, NWIN // 2, body, 0)
    wait_out()
    wait_out()


def _tc_body(ptr_ref, delays_ref, spikes_ref, buf_ref, out_ref):
    p = ptr_ref[0]
    d = delays_ref[...]
    r = (p + 1 - d) & (D_ROWS - 1)
    rows = jax.lax.broadcasted_iota(jnp.int32, (D_ROWS, BC), 0)
    hit = rows == r[None, :]
    val = jnp.sum(jnp.where(hit, buf_ref[...], 0.0), axis=0)
    out_ref[...] = jnp.where(r == (p & (D_ROWS - 1)), spikes_ref[...], val)


def _tc_delay_gather(buffer, delays_i, spikes_f, ptr_arr):
    return pl.pallas_call(
        _tc_body,
        grid=(NBLK,),
        in_specs=[
            pl.BlockSpec(memory_space=pltpu.SMEM),
            pl.BlockSpec((BC,), lambda i: (i + BOFF,)),
            pl.BlockSpec((BC,), lambda i: (i + BOFF,)),
            pl.BlockSpec((D_ROWS, BC), lambda i: (0, i + BOFF)),
        ],
        out_specs=pl.BlockSpec((BC,), lambda i: (i,)),
        out_shape=jax.ShapeDtypeStruct((NBLK * BC,), jnp.float32),
    )(ptr_arr, delays_i, spikes_f, buffer)


def kernel(buffer, spikes, delays, ptr):
    delays_i = delays.astype(jnp.int32)
    spikes_f = spikes.astype(jnp.float32)
    ptr_sc = jnp.full((L,), ptr, dtype=jnp.int32)
    ptr_tc = jnp.full((1,), ptr, dtype=jnp.int32)
    out_sc = _sc_delay_gather(buffer, delays_i, spikes_f, ptr_sc)
    out_tc = _tc_delay_gather(buffer, delays_i, spikes_f, ptr_tc)
    return jnp.concatenate([out_sc, out_tc[:SIZE - SC_COLS]])
